# Initial kernel scaffold; baseline (speedup 1.0000x reference)
import jax
import jax.numpy as jnp
from jax import lax
from jax.experimental import pallas as pl
from jax.experimental.pallas import tpu as pltpu
from jax.experimental.pallas import tpu_sc as plsc

NC = 2
NS = 16
NW = NC * NS

ROWS, COLS = 16384, 200
EMBED = 32
B = ROWS * COLS
B_PER_W = B // NW
CHUNK = 1024
STEPS = B_PER_W // CHUNK


def _gather_kernel(idx_hbm, table_hbm, out_hbm, idx_v, rows_v, sem):
    wid = lax.axis_index("s") * NC + lax.axis_index("c")
    base = wid * B_PER_W

    @pl.loop(0, STEPS)
    def _step(i):
        off = base + i * CHUNK
        pltpu.sync_copy(idx_hbm.at[pl.ds(off, CHUNK)], idx_v)
        pltpu.async_copy(table_hbm.at[idx_v], rows_v, sem).wait()
        pltpu.sync_copy(rows_v, out_hbm.at[pl.ds(off, CHUNK)])


@jax.jit
def _embed(idx_flat, table):
    run = pl.kernel(
        _gather_kernel,
        out_type=jax.ShapeDtypeStruct((B, EMBED), jnp.float32),
        mesh=plsc.VectorSubcoreMesh(
            core_axis_name="c", subcore_axis_name="s", num_cores=NC,
            num_subcores=NS,
        ),
        scratch_types=[
            pltpu.VMEM((CHUNK,), jnp.int32),
            pltpu.VMEM((CHUNK, EMBED), jnp.float32),
            pltpu.SemaphoreType.DMA,
        ],
    )
    return run(idx_flat, table)


def kernel(id, table):
    idx_flat = id.reshape(B).astype(jnp.int32)
    out = _embed(idx_flat, table)
    return out.reshape(ROWS, COLS, EMBED)


# speedup vs baseline: 4.8073x; 4.8073x over previous
import jax
import jax.numpy as jnp
from jax import lax
from jax.experimental import pallas as pl
from jax.experimental.pallas import tpu as pltpu
from jax.experimental.pallas import tpu_sc as plsc

NC = 2
NS = 16
NW = NC * NS

ROWS, COLS = 16384, 200
EMBED = 32
B = ROWS * COLS
B_PER_W = B // NW
CHUNK = 1024
STEPS = B_PER_W // CHUNK


def _gather_kernel(idx_hbm, table_hbm, out_hbm, idx_v, rows_v, sem):
    wid = lax.axis_index("s") * NC + lax.axis_index("c")
    base = wid * B_PER_W

    @pl.loop(0, STEPS)
    def _step(i):
        off = base + i * CHUNK
        pltpu.sync_copy(idx_hbm.at[pl.ds(off, CHUNK)], idx_v)
        pltpu.async_copy(table_hbm.at[idx_v], rows_v, sem).wait()
        pltpu.sync_copy(rows_v, out_hbm.at[pl.ds(off, CHUNK)])


@jax.jit
def _embed(idx_flat, table):
    run = pl.kernel(
        _gather_kernel,
        out_type=jax.ShapeDtypeStruct((B, EMBED), jnp.float32),
        mesh=plsc.VectorSubcoreMesh(
            core_axis_name="c", subcore_axis_name="s", num_cores=NC,
            num_subcores=NS,
        ),
        scratch_types=[
            pltpu.VMEM((CHUNK,), jnp.int32),
            pltpu.VMEM((CHUNK, EMBED), jnp.float32),
            pltpu.SemaphoreType.DMA,
        ],
        compiler_params=pltpu.CompilerParams(use_tc_tiling_on_sc=False),
    )
    return run(idx_flat, table)


def kernel(id, table):
    idx_flat = id.reshape(B).astype(jnp.int32)
    out = _embed(idx_flat, table)
    return out.reshape(ROWS, COLS, EMBED)


# speedup vs baseline: 5.0496x; 1.0504x over previous
import jax
import jax.numpy as jnp
from jax import lax
from jax.experimental import pallas as pl
from jax.experimental.pallas import tpu as pltpu
from jax.experimental.pallas import tpu_sc as plsc

NC = 2
NS = 16
NW = NC * NS

ROWS, COLS = 16384, 200
EMBED = 32
B = ROWS * COLS
B_PER_W = B // NW
CHUNK = 512
STEPS = B_PER_W // CHUNK
NBUF = 4


def _gather_kernel(idx_hbm, table_hbm, out_hbm, idx_v, rows_v,
                   sem_i, sem_g, sem_w):
    wid = lax.axis_index("s") * NC + lax.axis_index("c")
    base = wid * B_PER_W

    def idx_copy(i, b):
        return pltpu.make_async_copy(
            idx_hbm.at[pl.ds(base + i * CHUNK, CHUNK)], idx_v.at[b],
            sem_i.at[b])

    def gather(b):
        return pltpu.make_async_copy(
            table_hbm.at[idx_v.at[b]], rows_v.at[b], sem_g.at[b])

    def writeback(i, b):
        return pltpu.make_async_copy(
            rows_v.at[b], out_hbm.at[pl.ds(base + i * CHUNK, CHUNK)],
            sem_w.at[b])

    for b in range(NBUF):
        idx_copy(b, b).start()
    for b in range(NBUF):
        idx_copy(b, b).wait()
        gather(b).start()

    @pl.loop(0, STEPS, step=NBUF)
    def _outer(g):
        for b in range(NBUF):
            i = g + b
            gather(b).wait()
            writeback(i, b).start()
            j = i + NBUF

            @pl.when(j < STEPS)
            def _refill():
                idx_copy(j, b).start()
                idx_copy(j, b).wait()
                writeback(i, b).wait()
                gather(b).start()

    for b in range(NBUF):
        writeback(STEPS - NBUF + b, b).wait()


@jax.jit
def _embed(idx_flat, table):
    run = pl.kernel(
        _gather_kernel,
        out_type=jax.ShapeDtypeStruct((B, EMBED), jnp.float32),
        mesh=plsc.VectorSubcoreMesh(
            core_axis_name="c", subcore_axis_name="s", num_cores=NC,
            num_subcores=NS,
        ),
        scratch_types=[
            pltpu.VMEM((NBUF, CHUNK), jnp.int32),
            pltpu.VMEM((NBUF, CHUNK, EMBED), jnp.float32),
            pltpu.SemaphoreType.DMA((NBUF,)),
            pltpu.SemaphoreType.DMA((NBUF,)),
            pltpu.SemaphoreType.DMA((NBUF,)),
        ],
        compiler_params=pltpu.CompilerParams(use_tc_tiling_on_sc=False),
    )
    return run(idx_flat, table)


def kernel(id, table):
    idx_flat = id.reshape(B).astype(jnp.int32)
    out = _embed(idx_flat, table)
    return out.reshape(ROWS, COLS, EMBED)


# speedup vs baseline: 7.4439x; 1.4741x over previous
import jax
import jax.numpy as jnp
from jax import lax
from jax.experimental import pallas as pl
from jax.experimental.pallas import tpu as pltpu
from jax.experimental.pallas import tpu_sc as plsc

NC = 2
NS = 16
NW = NC * NS

ROWS, COLS = 16384, 200
EMBED = 32
B = ROWS * COLS
B_PER_W = B // NW
CHUNK = 512
STEPS = B_PER_W // CHUNK
NBUF = 4

BW = ROWS // NW
QW = BW // 4


def _gather_kernel(idx_hbm, table_hbm, out_hbm, idx_v, rows_v,
                   sem_i, sem_g, sem_w):
    wid = lax.axis_index("s") * NC + lax.axis_index("c")
    base = wid * B_PER_W

    def idx_copy(i, b):
        return pltpu.make_async_copy(
            idx_hbm.at[pl.ds(base + i * CHUNK, CHUNK)], idx_v.at[b],
            sem_i.at[b])

    def gather(b):
        return pltpu.make_async_copy(
            table_hbm.at[idx_v.at[b]], rows_v.at[b], sem_g.at[b])

    def writeback(i, b):
        return pltpu.make_async_copy(
            rows_v.at[b], out_hbm.at[pl.ds(base + i * CHUNK, CHUNK)],
            sem_w.at[b])

    for b in range(NBUF):
        idx_copy(b, b).start()
    for b in range(NBUF):
        idx_copy(b, b).wait()
        gather(b).start()

    @pl.loop(0, STEPS, step=NBUF)
    def _outer(g):
        for b in range(NBUF):
            i = g + b
            gather(b).wait()
            writeback(i, b).start()
            j = i + NBUF

            @pl.when(j < STEPS)
            def _refill():
                idx_copy(j, b).start()
                idx_copy(j, b).wait()
                writeback(i, b).wait()
                gather(b).start()

    for b in range(NBUF):
        writeback(STEPS - NBUF + b, b).wait()


def _transpose_kernel(inter_hbm, out_hbm, in_v, t_v, sem_l, sem_w):
    wid = lax.axis_index("s") * NC + lax.axis_index("c")
    lanes = jnp.arange(16, dtype=jnp.int32)
    colb = (lanes & 3) * 32

    def load(s, b):
        return pltpu.make_async_copy(
            inter_hbm.at[pl.ds(s * (ROWS // 4) + wid * QW, QW), :],
            in_v.at[b], sem_l.at[b])

    def wb(s, b, fb, bb):
        return pltpu.make_async_copy(
            t_v.at[b, fb, bb],
            out_hbm.at[s, pl.ds(8 * fb, 8),
                       pl.ds(wid * BW + 128 * bb, 128)],
            sem_w.at[b])

    def transpose(b):
        for f0 in (0, 16):
            @pl.loop(0, 16)
            def _d(d):
                f = f0 + ((lanes + d) & 15)
                col = colb + f
                fb = f >> 3
                fs = f & 7

                @pl.loop(0, BW, step=16, unroll=8)
                def _j(j0):
                    jv = j0 + lanes
                    v = plsc.load_gather(in_v.at[b], [jv >> 2, col])
                    plsc.store_scatter(
                        t_v.at[b], [fb, jv >> 7, fs, jv & 127], v)

    load(0, 0).start()
    load(1, 1).start()

    @pl.loop(0, COLS, step=2)
    def _outer(g):
        for b in range(2):
            s = g + b
            load(s, b).wait()

            @pl.when(s >= 2)
            def _drain():
                for fb in range(4):
                    for bb in range(4):
                        wb(s - 2, b, fb, bb).wait()

            transpose(b)

            @pl.when(s + 2 < COLS)
            def _next():
                load(s + 2, b).start()
            for fb in range(4):
                for bb in range(4):
                    wb(s, b, fb, bb).start()

    for b in range(2):
        for fb in range(4):
            for bb in range(4):
                wb(COLS - 2 + b, b, fb, bb).wait()


@jax.jit
def _embed(idx_flat, table):
    inter = pl.kernel(
        _gather_kernel,
        out_type=jax.ShapeDtypeStruct((B, EMBED), jnp.float32),
        mesh=plsc.VectorSubcoreMesh(
            core_axis_name="c", subcore_axis_name="s", num_cores=NC,
            num_subcores=NS,
        ),
        scratch_types=[
            pltpu.VMEM((NBUF, CHUNK), jnp.int32),
            pltpu.VMEM((NBUF, CHUNK, EMBED), jnp.float32),
            pltpu.SemaphoreType.DMA((NBUF,)),
            pltpu.SemaphoreType.DMA((NBUF,)),
            pltpu.SemaphoreType.DMA((NBUF,)),
        ],
        compiler_params=pltpu.CompilerParams(use_tc_tiling_on_sc=False),
    )(idx_flat, table)

    raw = pl.kernel(
        _transpose_kernel,
        out_type=jax.ShapeDtypeStruct((COLS, EMBED, ROWS), jnp.float32),
        mesh=plsc.VectorSubcoreMesh(
            core_axis_name="c", subcore_axis_name="s", num_cores=NC,
            num_subcores=NS,
        ),
        scratch_types=[
            pltpu.VMEM((2, QW, 128), jnp.float32),
            pltpu.VMEM((2, 4, 4, 8, 128), jnp.float32),
            pltpu.SemaphoreType.DMA((2,)),
            pltpu.SemaphoreType.DMA((2,)),
        ],
        compiler_params=pltpu.CompilerParams(
            use_tc_tiling_on_sc=True, needs_layout_passes=False),
    )(inter.reshape(B // 4, 128))
    return raw


def kernel(id, table):
    idx_flat = id.T.reshape(B).astype(jnp.int32)
    raw = _embed(idx_flat, table)
    return jnp.transpose(raw, (2, 0, 1))


# speedup vs baseline: 7.5998x; 1.0210x over previous
import jax
import jax.numpy as jnp
from jax import lax
from jax.experimental import pallas as pl
from jax.experimental.pallas import tpu as pltpu
from jax.experimental.pallas import tpu_sc as plsc

NC = 2
NS = 16
NW = NC * NS

ROWS, COLS = 16384, 200
EMBED = 32
B = ROWS * COLS
B_PER_W = B // NW
CHUNK = 512
STEPS = B_PER_W // CHUNK
NBUF = 4
CPR = ROWS // CHUNK

BW = ROWS // NW
QW = BW // 4


def _gather_kernel(idx_hbm, table_hbm, out_hbm, idx_v, rows_v,
                   sem_i, sem_g, sem_w):
    wid = lax.axis_index("s") * NC + lax.axis_index("c")
    base = wid * B_PER_W

    def idx_copy(i, b):
        c = wid * STEPS + i
        return pltpu.make_async_copy(
            idx_hbm.at[c // CPR, pl.ds((c % CPR) * CHUNK, CHUNK)],
            idx_v.at[b], sem_i.at[b])

    def gather(b):
        return pltpu.make_async_copy(
            table_hbm.at[idx_v.at[b]], rows_v.at[b], sem_g.at[b])

    def writeback(i, b):
        return pltpu.make_async_copy(
            rows_v.at[b], out_hbm.at[pl.ds(base + i * CHUNK, CHUNK)],
            sem_w.at[b])

    for b in range(NBUF):
        idx_copy(b, b).start()
    for b in range(NBUF):
        idx_copy(b, b).wait()
        gather(b).start()

    @pl.loop(0, STEPS, step=NBUF)
    def _outer(g):
        for b in range(NBUF):
            i = g + b
            gather(b).wait()
            writeback(i, b).start()
            j = i + NBUF

            @pl.when(j < STEPS)
            def _refill():
                idx_copy(j, b).start()
                idx_copy(j, b).wait()
                writeback(i, b).wait()
                gather(b).start()

    for b in range(NBUF):
        writeback(STEPS - NBUF + b, b).wait()


def _transpose_kernel(inter_hbm, out_hbm, in_v, t_v, sem_l, sem_w):
    wid = lax.axis_index("s") * NC + lax.axis_index("c")
    lanes = jnp.arange(16, dtype=jnp.int32)

    def load(s, b):
        return pltpu.make_async_copy(
            inter_hbm.at[pl.ds(s * (ROWS // 4) + wid * QW, QW), :],
            in_v.at[b], sem_l.at[b])

    def wb(s, b, fb, bb):
        return pltpu.make_async_copy(
            t_v.at[b, fb, bb],
            out_hbm.at[s, pl.ds(8 * fb, 8),
                       pl.ds(wid * BW + 128 * bb, 128)],
            sem_w.at[b])

    def transpose(b):
        in2 = in_v.at[b]
        t2 = t_v.at[b].reshape(16, BW * EMBED // 16)
        for f0 in (0, 16):
            @pl.loop(0, 16)
            def _d(d):
                f = f0 + ((lanes + d) & 15)
                col = (lanes & 3) * 32 + f
                row0 = lanes >> 2
                rv0 = (f >> 3) * 4 + 0
                cv0 = (f & 7) * 128 + lanes

                def _j(k, carry):
                    row, rv, cv = carry
                    for m in range(8):
                        v = plsc.load_gather(in2, [row, col])
                        plsc.store_scatter(t2, [rv, cv], v)
                        row = row + 4
                        if m == 7:
                            rv = rv + 1
                            cv = cv - 112
                        else:
                            cv = cv + 16
                    return (row, rv, cv)

                pl.loop(0, 4, init_carry=(row0, rv0, cv0))(_j)

    load(0, 0).start()
    load(1, 1).start()

    @pl.loop(0, COLS, step=2)
    def _outer(g):
        for b in range(2):
            s = g + b
            load(s, b).wait()

            @pl.when(s >= 2)
            def _drain():
                for fb in range(4):
                    for bb in range(4):
                        wb(s - 2, b, fb, bb).wait()

            transpose(b)

            @pl.when(s + 2 < COLS)
            def _next():
                load(s + 2, b).start()
            for fb in range(4):
                for bb in range(4):
                    wb(s, b, fb, bb).start()

    for b in range(2):
        for fb in range(4):
            for bb in range(4):
                wb(COLS - 2 + b, b, fb, bb).wait()


@jax.jit
def _embed(idx_t, table):
    inter = pl.kernel(
        _gather_kernel,
        out_type=jax.ShapeDtypeStruct((B, EMBED), jnp.float32),
        mesh=plsc.VectorSubcoreMesh(
            core_axis_name="c", subcore_axis_name="s", num_cores=NC,
            num_subcores=NS,
        ),
        scratch_types=[
            pltpu.VMEM((NBUF, CHUNK), jnp.int32),
            pltpu.VMEM((NBUF, CHUNK, EMBED), jnp.float32),
            pltpu.SemaphoreType.DMA((NBUF,)),
            pltpu.SemaphoreType.DMA((NBUF,)),
            pltpu.SemaphoreType.DMA((NBUF,)),
        ],
        compiler_params=pltpu.CompilerParams(use_tc_tiling_on_sc=False),
    )(idx_t, table)

    raw = pl.kernel(
        _transpose_kernel,
        out_type=jax.ShapeDtypeStruct((COLS, EMBED, ROWS), jnp.float32),
        mesh=plsc.VectorSubcoreMesh(
            core_axis_name="c", subcore_axis_name="s", num_cores=NC,
            num_subcores=NS,
        ),
        scratch_types=[
            pltpu.VMEM((2, QW, 128), jnp.float32),
            pltpu.VMEM((2, 4, 4, 8, 128), jnp.float32),
            pltpu.SemaphoreType.DMA((2,)),
            pltpu.SemaphoreType.DMA((2,)),
        ],
        compiler_params=pltpu.CompilerParams(
            use_tc_tiling_on_sc=True, needs_layout_passes=False),
    )(inter.reshape(B // 4, 128))
    return raw


def kernel(id, table):
    idx_t = id.T.astype(jnp.int32)
    raw = _embed(idx_t, table)
    return jnp.transpose(raw, (2, 0, 1))


# speedup vs baseline: 9.6204x; 1.2659x over previous
import jax
import jax.numpy as jnp
from jax import lax
from jax.experimental import pallas as pl
from jax.experimental.pallas import tpu as pltpu
from jax.experimental.pallas import tpu_sc as plsc

NC = 2
NS = 16
NW = NC * NS

ROWS, COLS = 16384, 200
EMBED = 32
B = ROWS * COLS
B_PER_W = B // NW
CHUNK = 512
STEPS = B_PER_W // CHUNK
NBUF = 4
CPR = ROWS // CHUNK

BW = ROWS // NW
QW = BW // 4


def _gather_kernel(idx_hbm, table_hbm, out_hbm, idx_v, rows_v,
                   sem_i, sem_g, sem_w):
    wid = lax.axis_index("s") * NC + lax.axis_index("c")
    base = wid * B_PER_W

    def idx_copy(i, b):
        c = wid * STEPS + i
        return pltpu.make_async_copy(
            idx_hbm.at[c // CPR, pl.ds((c % CPR) * CHUNK, CHUNK)],
            idx_v.at[b], sem_i.at[b])

    def gather(b):
        return pltpu.make_async_copy(
            table_hbm.at[idx_v.at[b]], rows_v.at[b], sem_g.at[b])

    def writeback(i, b):
        return pltpu.make_async_copy(
            rows_v.at[b], out_hbm.at[pl.ds(base + i * CHUNK, CHUNK)],
            sem_w.at[b])

    for b in range(NBUF):
        idx_copy(b, b).start()
    for b in range(NBUF):
        idx_copy(b, b).wait()
        gather(b).start()

    @pl.loop(0, STEPS, step=NBUF)
    def _outer(g):
        for b in range(NBUF):
            i = g + b
            gather(b).wait()
            writeback(i, b).start()
            j = i + NBUF

            @pl.when(j < STEPS)
            def _refill():
                idx_copy(j, b).start()
                idx_copy(j, b).wait()
                writeback(i, b).wait()
                gather(b).start()

    for b in range(NBUF):
        writeback(STEPS - NBUF + b, b).wait()


def _transpose_kernel(inter_hbm, out_hbm, in_v, t_v, sem_l, sem_w):
    wid = lax.axis_index("s") * NC + lax.axis_index("c")
    lanes = jnp.arange(16, dtype=jnp.int32)

    def load(s, b):
        return pltpu.make_async_copy(
            inter_hbm.at[pl.ds(s * (ROWS // 4) + wid * QW, QW), :],
            in_v.at[b], sem_l.at[b])

    def wb(s, b, fb, bb):
        return pltpu.make_async_copy(
            t_v.at[b, fb, bb],
            out_hbm.at[s, pl.ds(8 * fb, 8),
                       pl.ds(wid * BW + 128 * bb, 128)],
            sem_w.at[b])

    def transpose(b):
        in2 = in_v.at[b]
        t2 = t_v.at[b].reshape(16, BW * EMBED // 16)
        for f0 in (0, 16):
            @pl.loop(0, 16)
            def _d(d):
                f = f0 + ((lanes + d) & 15)
                col = (lanes & 3) * 32 + f
                rvb = (f >> 3) * 4
                cvb = (f & 7) * 128

                @plsc.parallel_loop(0, BW, 16, unroll=8)
                def _j(j0):
                    jv = j0 + lanes
                    v = plsc.load_gather(in2, [jv >> 2, col])
                    plsc.store_scatter(
                        t2, [rvb + (jv >> 7), cvb + (jv & 127)], v)

    load(0, 0).start()
    load(1, 1).start()

    @pl.loop(0, COLS, step=2)
    def _outer(g):
        for b in range(2):
            s = g + b
            load(s, b).wait()

            @pl.when(s >= 2)
            def _drain():
                for fb in range(4):
                    for bb in range(4):
                        wb(s - 2, b, fb, bb).wait()

            transpose(b)

            @pl.when(s + 2 < COLS)
            def _next():
                load(s + 2, b).start()
            for fb in range(4):
                for bb in range(4):
                    wb(s, b, fb, bb).start()

    for b in range(2):
        for fb in range(4):
            for bb in range(4):
                wb(COLS - 2 + b, b, fb, bb).wait()


@jax.jit
def _embed(idx_t, table):
    inter = pl.kernel(
        _gather_kernel,
        out_type=jax.ShapeDtypeStruct((B, EMBED), jnp.float32),
        mesh=plsc.VectorSubcoreMesh(
            core_axis_name="c", subcore_axis_name="s", num_cores=NC,
            num_subcores=NS,
        ),
        scratch_types=[
            pltpu.VMEM((NBUF, CHUNK), jnp.int32),
            pltpu.VMEM((NBUF, CHUNK, EMBED), jnp.float32),
            pltpu.SemaphoreType.DMA((NBUF,)),
            pltpu.SemaphoreType.DMA((NBUF,)),
            pltpu.SemaphoreType.DMA((NBUF,)),
        ],
        compiler_params=pltpu.CompilerParams(use_tc_tiling_on_sc=False),
    )(idx_t, table)

    raw = pl.kernel(
        _transpose_kernel,
        out_type=jax.ShapeDtypeStruct((COLS, EMBED, ROWS), jnp.float32),
        mesh=plsc.VectorSubcoreMesh(
            core_axis_name="c", subcore_axis_name="s", num_cores=NC,
            num_subcores=NS,
        ),
        scratch_types=[
            pltpu.VMEM((2, QW, 128), jnp.float32),
            pltpu.VMEM((2, 4, 4, 8, 128), jnp.float32),
            pltpu.SemaphoreType.DMA((2,)),
            pltpu.SemaphoreType.DMA((2,)),
        ],
        compiler_params=pltpu.CompilerParams(
            use_tc_tiling_on_sc=True, needs_layout_passes=False),
    )(inter.reshape(B // 4, 128))
    return raw


def kernel(id, table):
    idx_t = id.T.astype(jnp.int32)
    raw = _embed(idx_t, table)
    return jnp.transpose(raw, (2, 0, 1))
